# baseline (speedup 1.0000x reference)
import functools

import jax
import jax.numpy as jnp
from jax import lax
from jax.experimental import pallas as pl
from jax.experimental.pallas import tpu as pltpu
from jax.experimental.pallas import tpu_sc as plsc

_N = 10000
_NP = 10240
_E = 320000
_ET = _E + _N
_D = 128
_H1, _C1 = 8, 8
_HC = _H1 * _C1
_C2 = 40
_CHUNK = 2048
_ETP = 331776
_EHALF = _ETP // 2
_NCHUNK = _EHALF // _CHUNK

_BLK = 512
_NBLK = _NP // _BLK


def _tc1_body(x_ref, w1t_ref, asrc_ref, adst_ref, h1t_ref, as_ref, ad_ref):
    h = lax.dot_general(w1t_ref[...], x_ref[...], (((1,), (1,)), ((), ())),
                        preferred_element_type=jnp.float32)
    h1t_ref[...] = h
    as_ref[...] = jnp.dot(asrc_ref[...], h, preferred_element_type=jnp.float32)
    ad_ref[...] = jnp.dot(adst_ref[...], h, preferred_element_type=jnp.float32)


def _tc1(xp, w1t, asrc, adst):
    return pl.pallas_call(
        _tc1_body,
        grid=(_NBLK,),
        in_specs=[
            pl.BlockSpec((_BLK, _D), lambda i: (i, 0)),
            pl.BlockSpec((_HC, _D), lambda i: (0, 0)),
            pl.BlockSpec((_H1, _HC), lambda i: (0, 0)),
            pl.BlockSpec((_H1, _HC), lambda i: (0, 0)),
        ],
        out_specs=[
            pl.BlockSpec((_HC, _BLK), lambda i: (0, i)),
            pl.BlockSpec((_H1, _BLK), lambda i: (0, i)),
            pl.BlockSpec((_H1, _BLK), lambda i: (0, i)),
        ],
        out_shape=[
            jax.ShapeDtypeStruct((_HC, _NP), jnp.float32),
            jax.ShapeDtypeStruct((_H1, _NP), jnp.float32),
            jax.ShapeDtypeStruct((_H1, _NP), jnp.float32),
        ],
    )(xp, w1t, asrc, adst)


def _tc2_body(num_ref, den_ref, w2t_ref, att2_ref, b1_ref,
              h2t_ref, aa_ref):
    num = num_ref[0] + num_ref[1]
    den = den_ref[0] + den_ref[1]
    parts = [num[h * _C1:(h + 1) * _C1, :] / den[h:h + 1, :]
             for h in range(_H1)]
    o = jnp.concatenate(parts, axis=0) + b1_ref[...]
    o = jnp.where(o > 0.0, o, jnp.exp(o) - 1.0)
    h2 = jnp.dot(w2t_ref[...], o, preferred_element_type=jnp.float32)
    h2t_ref[...] = h2
    aa_ref[...] = jnp.dot(att2_ref[...], h2, preferred_element_type=jnp.float32)


def _tc2(numer1, denom1, w2t, att2, b1c):
    return pl.pallas_call(
        _tc2_body,
        grid=(_NBLK,),
        in_specs=[
            pl.BlockSpec((2, _HC, _BLK), lambda i: (0, 0, i)),
            pl.BlockSpec((2, _H1, _BLK), lambda i: (0, 0, i)),
            pl.BlockSpec((_C2, _HC), lambda i: (0, 0)),
            pl.BlockSpec((2, _C2), lambda i: (0, 0)),
            pl.BlockSpec((_HC, 1), lambda i: (0, 0)),
        ],
        out_specs=[
            pl.BlockSpec((_C2, _BLK), lambda i: (0, i)),
            pl.BlockSpec((2, _BLK), lambda i: (0, i)),
        ],
        out_shape=[
            jax.ShapeDtypeStruct((_C2, _NP), jnp.float32),
            jax.ShapeDtypeStruct((2, _NP), jnp.float32),
        ],
    )(numer1, denom1, w2t, att2, b1c)


def _tc3_body(num_ref, den_ref, b2_ref, out_ref):
    num = num_ref[0] + num_ref[1]
    den = den_ref[0] + den_ref[1]
    o = num / den + b2_ref[...]
    m = jnp.max(o, axis=0, keepdims=True)
    z = o - m
    lse = jnp.log(jnp.sum(jnp.exp(z), axis=0, keepdims=True))
    out_ref[...] = z - lse


def _tc3(numer2, denom2, b2c):
    return pl.pallas_call(
        _tc3_body,
        grid=(_NBLK,),
        in_specs=[
            pl.BlockSpec((2, _C2, _BLK), lambda i: (0, 0, i)),
            pl.BlockSpec((2, 1, _BLK), lambda i: (0, 0, i)),
            pl.BlockSpec((_C2, 1), lambda i: (0, 0)),
        ],
        out_specs=pl.BlockSpec((_C2, _BLK), lambda i: (0, i)),
        out_shape=jax.ShapeDtypeStruct((_C2, _NP), jnp.float32),
    )(numer2, denom2, b2c)


def _sweep(eidx_hbm, ebase, ebuf, tas, tad, tab, acc, accd,
           sem0, sem1, sem2, sem3):
    nch = len(tab)

    def _issue(c, slot, sem):
        off = pl.multiple_of(ebase + c * _CHUNK, 128)
        pltpu.async_copy(eidx_hbm.at[:, pl.ds(off, _CHUNK)],
                         ebuf.at[slot], sem)

    def _wait(slot, sem):
        pltpu.make_async_copy(eidx_hbm.at[:, pl.ds(0, _CHUNK)],
                              ebuf.at[slot], sem).wait()

    sems = (sem0, sem1, sem2, sem3)

    def _compute(slot):
        @plsc.parallel_loop(0, _CHUNK, 16, unroll=1)
        def _grp(b):
            bb = pl.multiple_of(b, 8)
            s16 = ebuf[slot, 0, pl.ds(bb, 16)]
            d16 = ebuf[slot, 1, pl.ds(bb, 16)]
            a = plsc.load_gather(tas, [s16]) + plsc.load_gather(tad, [d16])
            a = jnp.where(a >= 0.0, a, a * 0.2)
            e = jnp.exp(a)
            for j in range(nch):
                g = plsc.load_gather(tab[j], [s16])
                plsc.addupdate_scatter(acc[j], [d16], e * g)
            plsc.addupdate_scatter(accd, [d16], e)

    for k in range(4):
        _issue(k, k, sems[k])

    zero = jnp.zeros((16,), jnp.float32)

    @plsc.parallel_loop(0, _NP, 16, unroll=8)
    def _zero(i):
        off = pl.multiple_of(i, 8)
        for j in range(nch):
            acc[j][pl.ds(off, 16)] = zero
        accd[pl.ds(off, 16)] = zero

    @pl.loop(0, _NCHUNK // 4)
    def _quad(t):
        c = 4 * t
        for k in range(4):
            _wait(k, sems[k])
            _compute(k)

            @pl.when(c + k + 4 < _NCHUNK)
            def _():
                _issue(c + k + 4, k, sems[k])

    for c in range((_NCHUNK // 4) * 4, _NCHUNK):
        k = c % 4
        _wait(k, sems[k])
        _compute(k)


def _sc1_body(eidx_hbm, h1t_hbm, as_hbm, ad_hbm,
              numer_hbm, denom_hbm,
              tab0, tab1, tab2, tab3, acc0, acc1, acc2, acc3,
              tas, tad, accd, ebuf, sem0, sem1, sem2, sem3):
    core = lax.axis_index("c")
    sid = lax.axis_index("s")
    tab = (tab0, tab1, tab2, tab3)
    acc = (acc0, acc1, acc2, acc3)
    c0 = sid * 4
    head = sid // 2
    for j in range(4):
        pltpu.sync_copy(h1t_hbm.at[c0 + j], tab[j])
    pltpu.sync_copy(as_hbm.at[head], tas)
    pltpu.sync_copy(ad_hbm.at[head], tad)

    ebase = core * _EHALF
    _sweep(eidx_hbm, ebase, ebuf, tas, tad, tab, acc, accd,
           sem0, sem1, sem2, sem3)

    for j in range(4):
        pltpu.sync_copy(acc[j], numer_hbm.at[core, c0 + j])

    @pl.when(sid % 2 == 0)
    def _():
        pltpu.sync_copy(accd, denom_hbm.at[core, head])


def _sc2_body(eidx_hbm, h2t_hbm, aa_hbm,
              numer_hbm, denom_hbm,
              tab0, tab1, tab2, acc0, acc1, acc2,
              tas, tad, accd, ebuf, sem0, sem1, sem2, sem3):
    core = lax.axis_index("c")
    sid = lax.axis_index("s")
    tab = (tab0, tab1, tab2)
    acc = (acc0, acc1, acc2)
    c0 = jnp.minimum(sid * 3, _C2 - 3)
    for j in range(3):
        pltpu.sync_copy(h2t_hbm.at[c0 + j], tab[j])
    pltpu.sync_copy(aa_hbm.at[0], tas)
    pltpu.sync_copy(aa_hbm.at[1], tad)

    ebase = core * _EHALF
    _sweep(eidx_hbm, ebase, ebuf, tas, tad, tab, acc, accd,
           sem0, sem1, sem2, sem3)

    for j in range(3):
        pltpu.sync_copy(acc[j], numer_hbm.at[core, c0 + j])

    @pl.when(sid == 15)
    def _():
        pltpu.sync_copy(accd, denom_hbm.at[core, 0])


def _sc_scratch(nch):
    rows = [pltpu.VMEM((_NP,), jnp.float32)] * (2 * nch)
    return rows + [
        pltpu.VMEM((_NP,), jnp.float32),
        pltpu.VMEM((_NP,), jnp.float32),
        pltpu.VMEM((_NP,), jnp.float32),
        pltpu.VMEM((4, 2, _CHUNK), jnp.int32),
        pltpu.SemaphoreType.DMA,
        pltpu.SemaphoreType.DMA,
        pltpu.SemaphoreType.DMA,
        pltpu.SemaphoreType.DMA,
    ]


def _sc1(eidx, h1T, asT, adT):
    mesh = plsc.VectorSubcoreMesh(core_axis_name="c", subcore_axis_name="s")
    return pl.kernel(
        _sc1_body,
        out_type=[
            jax.ShapeDtypeStruct((2, _HC, _NP), jnp.float32),
            jax.ShapeDtypeStruct((2, _H1, _NP), jnp.float32),
        ],
        mesh=mesh,
        scratch_types=_sc_scratch(4),
        compiler_params=pltpu.CompilerParams(needs_layout_passes=False),
    )(eidx, h1T, asT, adT)


def _sc2(eidx, h2T, aaT):
    mesh = plsc.VectorSubcoreMesh(core_axis_name="c", subcore_axis_name="s")
    return pl.kernel(
        _sc2_body,
        out_type=[
            jax.ShapeDtypeStruct((2, _C2, _NP), jnp.float32),
            jax.ShapeDtypeStruct((2, 1, _NP), jnp.float32),
        ],
        mesh=mesh,
        scratch_types=_sc_scratch(3),
        compiler_params=pltpu.CompilerParams(needs_layout_passes=False),
    )(eidx, h2T, aaT)


@jax.jit
def kernel(x, edge_index, W1, att_src1, att_dst1, b1, W2, att_src2,
           att_dst2, b2):
    loops = jnp.arange(_N, dtype=edge_index.dtype)
    loops2 = jnp.stack([loops, loops])
    padE = jnp.full((2, _ETP - _ET), _N, dtype=edge_index.dtype)
    eidx = jnp.concatenate([edge_index, loops2, padE], axis=1)

    xp = jnp.pad(x, ((0, _NP - _N), (0, 0)))
    w1t = W1.T
    eye = jnp.eye(_H1, dtype=jnp.float32)
    asrc = (eye[:, :, None] * att_src1[:, None, :]).reshape(_H1, _HC)
    adst = (eye[:, :, None] * att_dst1[:, None, :]).reshape(_H1, _HC)

    h1T, asT, adT = _tc1(xp, w1t, asrc, adst)
    numer1, denom1 = _sc1(eidx, h1T, asT, adT)

    w2t = W2.T
    att2 = jnp.concatenate([att_src2, att_dst2], axis=0)
    h2T, aaT = _tc2(numer1, denom1, w2t, att2, b1.reshape(_HC, 1))

    numer2, denom2 = _sc2(eidx, h2T, aaT)
    outT = _tc3(numer2, denom2, b2.reshape(_C2, 1))
    return outT[:, :_N].T


# speedup vs baseline: 1.1002x; 1.1002x over previous
import functools

import jax
import jax.numpy as jnp
from jax import lax
from jax.experimental import pallas as pl
from jax.experimental.pallas import tpu as pltpu
from jax.experimental.pallas import tpu_sc as plsc

_N = 10000
_NP = 10240
_E = 320000
_ET = _E + _N
_D = 128
_H1, _C1 = 8, 8
_HC = _H1 * _C1
_C2 = 40
_CHUNK = 2048
_ETP = 331776
_EHALF = _ETP // 2
_NCHUNK = _EHALF // _CHUNK

_BLK = 512
_NBLK = _NP // _BLK


def _tc1_body(x_ref, w1t_ref, asrc_ref, adst_ref, h1t_ref, as_ref, ad_ref):
    h = lax.dot_general(w1t_ref[...], x_ref[...], (((1,), (1,)), ((), ())),
                        preferred_element_type=jnp.float32)
    h1t_ref[...] = h
    as_ref[...] = jnp.dot(asrc_ref[...], h, preferred_element_type=jnp.float32)
    ad_ref[...] = jnp.dot(adst_ref[...], h, preferred_element_type=jnp.float32)


def _tc1(xp, w1t, asrc, adst):
    return pl.pallas_call(
        _tc1_body,
        grid=(_NBLK,),
        in_specs=[
            pl.BlockSpec((_BLK, _D), lambda i: (i, 0)),
            pl.BlockSpec((_HC, _D), lambda i: (0, 0)),
            pl.BlockSpec((_H1, _HC), lambda i: (0, 0)),
            pl.BlockSpec((_H1, _HC), lambda i: (0, 0)),
        ],
        out_specs=[
            pl.BlockSpec((_HC, _BLK), lambda i: (0, i)),
            pl.BlockSpec((_H1, _BLK), lambda i: (0, i)),
            pl.BlockSpec((_H1, _BLK), lambda i: (0, i)),
        ],
        out_shape=[
            jax.ShapeDtypeStruct((_HC, _NP), jnp.float32),
            jax.ShapeDtypeStruct((_H1, _NP), jnp.float32),
            jax.ShapeDtypeStruct((_H1, _NP), jnp.float32),
        ],
    )(xp, w1t, asrc, adst)


def _tc2_body(num_ref, den_ref, w2t_ref, att2_ref, b1_ref,
              h2t_ref, aa_ref):
    num = num_ref[0] + num_ref[1]
    den = den_ref[0] + den_ref[1]
    parts = [num[h * _C1:(h + 1) * _C1, :] / den[h:h + 1, :]
             for h in range(_H1)]
    o = jnp.concatenate(parts, axis=0) + b1_ref[...]
    o = jnp.where(o > 0.0, o, jnp.exp(o) - 1.0)
    h2 = jnp.dot(w2t_ref[...], o, preferred_element_type=jnp.float32)
    h2t_ref[...] = h2
    aa_ref[...] = jnp.dot(att2_ref[...], h2, preferred_element_type=jnp.float32)


def _tc2(numer1, denom1, w2t, att2, b1c):
    return pl.pallas_call(
        _tc2_body,
        grid=(_NBLK,),
        in_specs=[
            pl.BlockSpec((2, _HC, _BLK), lambda i: (0, 0, i)),
            pl.BlockSpec((2, _H1, _BLK), lambda i: (0, 0, i)),
            pl.BlockSpec((_C2, _HC), lambda i: (0, 0)),
            pl.BlockSpec((2, _C2), lambda i: (0, 0)),
            pl.BlockSpec((_HC, 1), lambda i: (0, 0)),
        ],
        out_specs=[
            pl.BlockSpec((_C2, _BLK), lambda i: (0, i)),
            pl.BlockSpec((2, _BLK), lambda i: (0, i)),
        ],
        out_shape=[
            jax.ShapeDtypeStruct((_C2, _NP), jnp.float32),
            jax.ShapeDtypeStruct((2, _NP), jnp.float32),
        ],
    )(numer1, denom1, w2t, att2, b1c)


def _tc3_body(num_ref, den_ref, b2_ref, out_ref):
    num = num_ref[0] + num_ref[1]
    den = den_ref[0] + den_ref[1]
    o = num / den + b2_ref[...]
    m = jnp.max(o, axis=0, keepdims=True)
    z = o - m
    lse = jnp.log(jnp.sum(jnp.exp(z), axis=0, keepdims=True))
    out_ref[...] = z - lse


def _tc3(numer2, denom2, b2c):
    return pl.pallas_call(
        _tc3_body,
        grid=(_NBLK,),
        in_specs=[
            pl.BlockSpec((2, _C2, _BLK), lambda i: (0, 0, i)),
            pl.BlockSpec((2, 1, _BLK), lambda i: (0, 0, i)),
            pl.BlockSpec((_C2, 1), lambda i: (0, 0)),
        ],
        out_specs=pl.BlockSpec((_C2, _BLK), lambda i: (0, i)),
        out_shape=jax.ShapeDtypeStruct((_C2, _NP), jnp.float32),
    )(numer2, denom2, b2c)


def _sweep(eidx_hbm, ebase, ebuf, tas, tad, tab, acc, accd,
           sem0, sem1, sem2, sem3):
    nch = len(tab)

    def _issue(c, slot, sem):
        off = pl.multiple_of(ebase + c * _CHUNK, 128)
        pltpu.async_copy(eidx_hbm.at[:, pl.ds(off, _CHUNK)],
                         ebuf.at[slot], sem)

    def _wait(slot, sem):
        pltpu.make_async_copy(eidx_hbm.at[:, pl.ds(0, _CHUNK)],
                              ebuf.at[slot], sem).wait()

    sems = (sem0, sem1, sem2, sem3)

    def _compute(slot):
        @plsc.parallel_loop(0, _CHUNK, 16, unroll=2)
        def _grp(b):
            bb = pl.multiple_of(b, 8)
            s16 = ebuf[slot, 0, pl.ds(bb, 16)]
            d16 = ebuf[slot, 1, pl.ds(bb, 16)]
            a = plsc.load_gather(tas, [s16]) + plsc.load_gather(tad, [d16])
            a = jnp.maximum(a, a * 0.2)
            e = jnp.exp(a)
            for j in range(nch):
                g = plsc.load_gather(tab[j], [s16])
                plsc.addupdate_scatter(acc[j], [d16], e * g)
            plsc.addupdate_scatter(accd, [d16], e)

    for k in range(4):
        _issue(k, k, sems[k])

    zero = jnp.zeros((16,), jnp.float32)

    @plsc.parallel_loop(0, _NP, 16, unroll=8)
    def _zero(i):
        off = pl.multiple_of(i, 8)
        for j in range(nch):
            acc[j][pl.ds(off, 16)] = zero
        accd[pl.ds(off, 16)] = zero

    @pl.loop(0, _NCHUNK // 4)
    def _quad(t):
        c = 4 * t
        for k in range(4):
            _wait(k, sems[k])
            _compute(k)

            @pl.when(c + k + 4 < _NCHUNK)
            def _():
                _issue(c + k + 4, k, sems[k])

    for c in range((_NCHUNK // 4) * 4, _NCHUNK):
        k = c % 4
        _wait(k, sems[k])
        _compute(k)


def _sc1_body(eidx_hbm, h1t_hbm, as_hbm, ad_hbm,
              numer_hbm, denom_hbm,
              tab0, tab1, tab2, tab3, acc0, acc1, acc2, acc3,
              tas, tad, accd, ebuf, sem0, sem1, sem2, sem3):
    core = lax.axis_index("c")
    sid = lax.axis_index("s")
    tab = (tab0, tab1, tab2, tab3)
    acc = (acc0, acc1, acc2, acc3)
    c0 = sid * 4
    head = sid // 2
    for j in range(4):
        pltpu.sync_copy(h1t_hbm.at[c0 + j], tab[j])
    pltpu.sync_copy(as_hbm.at[head], tas)
    pltpu.sync_copy(ad_hbm.at[head], tad)

    ebase = core * _EHALF
    _sweep(eidx_hbm, ebase, ebuf, tas, tad, tab, acc, accd,
           sem0, sem1, sem2, sem3)

    for j in range(4):
        pltpu.sync_copy(acc[j], numer_hbm.at[core, c0 + j])

    @pl.when(sid % 2 == 0)
    def _():
        pltpu.sync_copy(accd, denom_hbm.at[core, head])


def _sc2_body(eidx_hbm, h2t_hbm, aa_hbm,
              numer_hbm, denom_hbm,
              tab0, tab1, tab2, acc0, acc1, acc2,
              tas, tad, accd, ebuf, sem0, sem1, sem2, sem3):
    core = lax.axis_index("c")
    sid = lax.axis_index("s")
    tab = (tab0, tab1, tab2)
    acc = (acc0, acc1, acc2)
    c0 = jnp.minimum(sid * 3, _C2 - 3)
    for j in range(3):
        pltpu.sync_copy(h2t_hbm.at[c0 + j], tab[j])
    pltpu.sync_copy(aa_hbm.at[0], tas)
    pltpu.sync_copy(aa_hbm.at[1], tad)

    ebase = core * _EHALF
    _sweep(eidx_hbm, ebase, ebuf, tas, tad, tab, acc, accd,
           sem0, sem1, sem2, sem3)

    for j in range(3):
        pltpu.sync_copy(acc[j], numer_hbm.at[core, c0 + j])

    @pl.when(sid == 15)
    def _():
        pltpu.sync_copy(accd, denom_hbm.at[core, 0])


def _sc_scratch(nch):
    rows = [pltpu.VMEM((_NP,), jnp.float32)] * (2 * nch)
    return rows + [
        pltpu.VMEM((_NP,), jnp.float32),
        pltpu.VMEM((_NP,), jnp.float32),
        pltpu.VMEM((_NP,), jnp.float32),
        pltpu.VMEM((4, 2, _CHUNK), jnp.int32),
        pltpu.SemaphoreType.DMA,
        pltpu.SemaphoreType.DMA,
        pltpu.SemaphoreType.DMA,
        pltpu.SemaphoreType.DMA,
    ]


def _sc1(eidx, h1T, asT, adT):
    mesh = plsc.VectorSubcoreMesh(core_axis_name="c", subcore_axis_name="s")
    return pl.kernel(
        _sc1_body,
        out_type=[
            jax.ShapeDtypeStruct((2, _HC, _NP), jnp.float32),
            jax.ShapeDtypeStruct((2, _H1, _NP), jnp.float32),
        ],
        mesh=mesh,
        scratch_types=_sc_scratch(4),
        compiler_params=pltpu.CompilerParams(needs_layout_passes=False),
    )(eidx, h1T, asT, adT)


def _sc2(eidx, h2T, aaT):
    mesh = plsc.VectorSubcoreMesh(core_axis_name="c", subcore_axis_name="s")
    return pl.kernel(
        _sc2_body,
        out_type=[
            jax.ShapeDtypeStruct((2, _C2, _NP), jnp.float32),
            jax.ShapeDtypeStruct((2, 1, _NP), jnp.float32),
        ],
        mesh=mesh,
        scratch_types=_sc_scratch(3),
        compiler_params=pltpu.CompilerParams(needs_layout_passes=False),
    )(eidx, h2T, aaT)


@jax.jit
def kernel(x, edge_index, W1, att_src1, att_dst1, b1, W2, att_src2,
           att_dst2, b2):
    loops = jnp.arange(_N, dtype=edge_index.dtype)
    loops2 = jnp.stack([loops, loops])
    padE = jnp.full((2, _ETP - _ET), _N, dtype=edge_index.dtype)
    eidx = jnp.concatenate([edge_index, loops2, padE], axis=1)

    xp = jnp.pad(x, ((0, _NP - _N), (0, 0)))
    w1t = W1.T
    eye = jnp.eye(_H1, dtype=jnp.float32)
    asrc = (eye[:, :, None] * att_src1[:, None, :]).reshape(_H1, _HC)
    adst = (eye[:, :, None] * att_dst1[:, None, :]).reshape(_H1, _HC)

    h1T, asT, adT = _tc1(xp, w1t, asrc, adst)
    numer1, denom1 = _sc1(eidx, h1T, asT, adT)

    w2t = W2.T
    att2 = jnp.concatenate([att_src2, att_dst2], axis=0)
    h2T, aaT = _tc2(numer1, denom1, w2t, att2, b1.reshape(_HC, 1))

    numer2, denom2 = _sc2(eidx, h2T, aaT)
    outT = _tc3(numer2, denom2, b2.reshape(_C2, 1))
    return outT[:, :_N].T


# speedup vs baseline: 1.1476x; 1.0431x over previous
import functools

import jax
import jax.numpy as jnp
from jax import lax
from jax.experimental import pallas as pl
from jax.experimental.pallas import tpu as pltpu
from jax.experimental.pallas import tpu_sc as plsc

_N = 10000
_NP = 10240
_E = 320000
_ET = _E + _N
_D = 128
_H1, _C1 = 8, 8
_HC = _H1 * _C1
_C2 = 40
_CHUNK = 2048
_ETP = 331776
_EHALF = _ETP // 2
_NCHUNK = _EHALF // _CHUNK

_BLK = 512
_NBLK = _NP // _BLK


def _tc1_body(x_ref, w1t_ref, asrc_ref, adst_ref, h1t_ref, as_ref, ad_ref):
    h = lax.dot_general(w1t_ref[...], x_ref[...], (((1,), (1,)), ((), ())),
                        preferred_element_type=jnp.float32)
    h1t_ref[...] = h
    as_ref[...] = jnp.dot(asrc_ref[...], h, preferred_element_type=jnp.float32)
    ad_ref[...] = jnp.dot(adst_ref[...], h, preferred_element_type=jnp.float32)


def _tc1(xp, w1t, asrc, adst):
    return pl.pallas_call(
        _tc1_body,
        grid=(_NBLK,),
        in_specs=[
            pl.BlockSpec((_BLK, _D), lambda i: (i, 0)),
            pl.BlockSpec((_HC, _D), lambda i: (0, 0)),
            pl.BlockSpec((_H1, _HC), lambda i: (0, 0)),
            pl.BlockSpec((_H1, _HC), lambda i: (0, 0)),
        ],
        out_specs=[
            pl.BlockSpec((_HC, _BLK), lambda i: (0, i)),
            pl.BlockSpec((_H1, _BLK), lambda i: (0, i)),
            pl.BlockSpec((_H1, _BLK), lambda i: (0, i)),
        ],
        out_shape=[
            jax.ShapeDtypeStruct((_HC, _NP), jnp.float32),
            jax.ShapeDtypeStruct((_H1, _NP), jnp.float32),
            jax.ShapeDtypeStruct((_H1, _NP), jnp.float32),
        ],
    )(xp, w1t, asrc, adst)


def _tc2_body(num_ref, den_ref, w2t_ref, att2_ref, b1_ref,
              h2t_ref, aa_ref):
    num = num_ref[0] + num_ref[1]
    den = den_ref[0] + den_ref[1]
    parts = [num[h * _C1:(h + 1) * _C1, :] / den[h:h + 1, :]
             for h in range(_H1)]
    o = jnp.concatenate(parts, axis=0) + b1_ref[...]
    o = jnp.where(o > 0.0, o, jnp.exp(o) - 1.0)
    h2 = jnp.dot(w2t_ref[...], o, preferred_element_type=jnp.float32)
    h2t_ref[...] = h2
    aa_ref[...] = jnp.dot(att2_ref[...], h2, preferred_element_type=jnp.float32)


def _tc2(numer1, denom1, w2t, att2, b1c):
    return pl.pallas_call(
        _tc2_body,
        grid=(_NBLK,),
        in_specs=[
            pl.BlockSpec((2, _HC, _BLK), lambda i: (0, 0, i)),
            pl.BlockSpec((2, _H1, _BLK), lambda i: (0, 0, i)),
            pl.BlockSpec((_C2, _HC), lambda i: (0, 0)),
            pl.BlockSpec((2, _C2), lambda i: (0, 0)),
            pl.BlockSpec((_HC, 1), lambda i: (0, 0)),
        ],
        out_specs=[
            pl.BlockSpec((_C2, _BLK), lambda i: (0, i)),
            pl.BlockSpec((2, _BLK), lambda i: (0, i)),
        ],
        out_shape=[
            jax.ShapeDtypeStruct((_C2, _NP), jnp.float32),
            jax.ShapeDtypeStruct((2, _NP), jnp.float32),
        ],
    )(numer1, denom1, w2t, att2, b1c)


def _tc3_body(num_ref, den_ref, b2_ref, out_ref):
    num = num_ref[0] + num_ref[1]
    den = den_ref[0] + den_ref[1]
    o = num / den + b2_ref[...]
    m = jnp.max(o, axis=0, keepdims=True)
    z = o - m
    lse = jnp.log(jnp.sum(jnp.exp(z), axis=0, keepdims=True))
    out_ref[...] = z - lse


def _tc3(numer2, denom2, b2c):
    return pl.pallas_call(
        _tc3_body,
        grid=(_NBLK,),
        in_specs=[
            pl.BlockSpec((2, _C2, _BLK), lambda i: (0, 0, i)),
            pl.BlockSpec((2, 1, _BLK), lambda i: (0, 0, i)),
            pl.BlockSpec((_C2, 1), lambda i: (0, 0)),
        ],
        out_specs=pl.BlockSpec((_C2, _BLK), lambda i: (0, i)),
        out_shape=jax.ShapeDtypeStruct((_C2, _NP), jnp.float32),
    )(numer2, denom2, b2c)


def _sweep(eidx_hbm, ebase, ebuf, tas, tad, tab, acc, accd,
           sem0, sem1, sem2, sem3):
    nch = len(tab)

    def _issue(c, slot, sem):
        off = pl.multiple_of(ebase + c * _CHUNK, 128)
        pltpu.async_copy(eidx_hbm.at[:, pl.ds(off, _CHUNK)],
                         ebuf.at[slot], sem)

    def _wait(slot, sem):
        pltpu.make_async_copy(eidx_hbm.at[:, pl.ds(0, _CHUNK)],
                              ebuf.at[slot], sem).wait()

    sems = (sem0, sem1, sem2, sem3)

    def _compute(slot):
        @plsc.parallel_loop(0, _CHUNK, 16, unroll=2)
        def _grp(b):
            bb = pl.multiple_of(b, 8)
            s16 = ebuf[slot, 0, pl.ds(bb, 16)]
            d16 = ebuf[slot, 1, pl.ds(bb, 16)]
            a = plsc.load_gather(tas, [s16]) + plsc.load_gather(tad, [d16])
            a = jnp.maximum(a, a * 0.2)
            e = jnp.exp(a)
            for j in range(nch):
                g = plsc.load_gather(tab[j], [s16])
                plsc.addupdate_scatter(acc[j], [d16], e * g)
            if accd is not None:
                plsc.addupdate_scatter(accd, [d16], e)

    for k in range(4):
        _issue(k, k, sems[k])

    zero = jnp.zeros((16,), jnp.float32)

    @plsc.parallel_loop(0, _NP, 16, unroll=8)
    def _zero(i):
        off = pl.multiple_of(i, 8)
        for j in range(nch):
            acc[j][pl.ds(off, 16)] = zero
        if accd is not None:
            accd[pl.ds(off, 16)] = zero

    @pl.loop(0, _NCHUNK // 4)
    def _quad(t):
        c = 4 * t
        for k in range(4):
            _wait(k, sems[k])
            _compute(k)

            @pl.when(c + k + 4 < _NCHUNK)
            def _():
                _issue(c + k + 4, k, sems[k])

    for c in range((_NCHUNK // 4) * 4, _NCHUNK):
        k = c % 4
        _wait(k, sems[k])
        _compute(k)


def _sc1_body(eidx_hbm, h1t_hbm, as_hbm, ad_hbm,
              numer_hbm, denom_hbm,
              tab0, tab1, tab2, tab3, acc0, acc1, acc2, acc3,
              tas, tad, accd, ebuf, sem0, sem1, sem2, sem3):
    core = lax.axis_index("c")
    sid = lax.axis_index("s")
    tab = (tab0, tab1, tab2, tab3)
    acc = (acc0, acc1, acc2, acc3)
    c0 = sid * 4
    head = sid // 2
    for j in range(4):
        pltpu.sync_copy(h1t_hbm.at[c0 + j], tab[j])
    pltpu.sync_copy(as_hbm.at[head], tas)
    pltpu.sync_copy(ad_hbm.at[head], tad)

    ebase = core * _EHALF
    _sweep(eidx_hbm, ebase, ebuf, tas, tad, tab, acc, accd,
           sem0, sem1, sem2, sem3)

    for j in range(4):
        pltpu.sync_copy(acc[j], numer_hbm.at[core, c0 + j])

    @pl.when(sid % 2 == 0)
    def _():
        pltpu.sync_copy(accd, denom_hbm.at[core, head])


def _sc2_body(eidx_hbm, h2t_hbm, aa_hbm,
              numer_hbm, denom_hbm,
              tab0, tab1, tab2, acc0, acc1, acc2,
              tas, tad, accd, ebuf, sem0, sem1, sem2, sem3):
    core = lax.axis_index("c")
    sid = lax.axis_index("s")
    pltpu.sync_copy(aa_hbm.at[0], tas)
    pltpu.sync_copy(aa_hbm.at[1], tad)
    ebase = core * _EHALF

    @pl.when(sid < 8)
    def _():
        c0 = sid * 3
        for j in range(3):
            pltpu.sync_copy(h2t_hbm.at[c0 + j], tab0 if j == 0 else
                            (tab1 if j == 1 else tab2))
        _sweep(eidx_hbm, ebase, ebuf, tas, tad, (tab0, tab1, tab2),
               (acc0, acc1, acc2), None, sem0, sem1, sem2, sem3)
        for j, aj in enumerate((acc0, acc1, acc2)):
            pltpu.sync_copy(aj, numer_hbm.at[core, c0 + j])

    @pl.when(sid >= 8)
    def _():
        c0 = 8 + sid * 2
        pltpu.sync_copy(h2t_hbm.at[c0], tab0)
        pltpu.sync_copy(h2t_hbm.at[c0 + 1], tab1)
        _sweep(eidx_hbm, ebase, ebuf, tas, tad, (tab0, tab1),
               (acc0, acc1), accd, sem0, sem1, sem2, sem3)
        pltpu.sync_copy(acc0, numer_hbm.at[core, c0])
        pltpu.sync_copy(acc1, numer_hbm.at[core, c0 + 1])

        @pl.when(sid == 15)
        def _():
            pltpu.sync_copy(accd, denom_hbm.at[core, 0])


def _sc_scratch(nch):
    rows = [pltpu.VMEM((_NP,), jnp.float32)] * (2 * nch)
    return rows + [
        pltpu.VMEM((_NP,), jnp.float32),
        pltpu.VMEM((_NP,), jnp.float32),
        pltpu.VMEM((_NP,), jnp.float32),
        pltpu.VMEM((4, 2, _CHUNK), jnp.int32),
        pltpu.SemaphoreType.DMA,
        pltpu.SemaphoreType.DMA,
        pltpu.SemaphoreType.DMA,
        pltpu.SemaphoreType.DMA,
    ]


def _sc1(eidx, h1T, asT, adT):
    mesh = plsc.VectorSubcoreMesh(core_axis_name="c", subcore_axis_name="s")
    return pl.kernel(
        _sc1_body,
        out_type=[
            jax.ShapeDtypeStruct((2, _HC, _NP), jnp.float32),
            jax.ShapeDtypeStruct((2, _H1, _NP), jnp.float32),
        ],
        mesh=mesh,
        scratch_types=_sc_scratch(4),
        compiler_params=pltpu.CompilerParams(needs_layout_passes=False),
    )(eidx, h1T, asT, adT)


def _sc2(eidx, h2T, aaT):
    mesh = plsc.VectorSubcoreMesh(core_axis_name="c", subcore_axis_name="s")
    return pl.kernel(
        _sc2_body,
        out_type=[
            jax.ShapeDtypeStruct((2, _C2, _NP), jnp.float32),
            jax.ShapeDtypeStruct((2, 1, _NP), jnp.float32),
        ],
        mesh=mesh,
        scratch_types=_sc_scratch(3),
        compiler_params=pltpu.CompilerParams(needs_layout_passes=False),
    )(eidx, h2T, aaT)


@jax.jit
def kernel(x, edge_index, W1, att_src1, att_dst1, b1, W2, att_src2,
           att_dst2, b2):
    loops = jnp.arange(_N, dtype=edge_index.dtype)
    loops2 = jnp.stack([loops, loops])
    padE = jnp.full((2, _ETP - _ET), _N, dtype=edge_index.dtype)
    eidx = jnp.concatenate([edge_index, loops2, padE], axis=1)

    xp = jnp.pad(x, ((0, _NP - _N), (0, 0)))
    w1t = W1.T
    eye = jnp.eye(_H1, dtype=jnp.float32)
    asrc = (eye[:, :, None] * att_src1[:, None, :]).reshape(_H1, _HC)
    adst = (eye[:, :, None] * att_dst1[:, None, :]).reshape(_H1, _HC)

    h1T, asT, adT = _tc1(xp, w1t, asrc, adst)
    numer1, denom1 = _sc1(eidx, h1T, asT, adT)

    w2t = W2.T
    att2 = jnp.concatenate([att_src2, att_dst2], axis=0)
    h2T, aaT = _tc2(numer1, denom1, w2t, att2, b1.reshape(_HC, 1))

    numer2, denom2 = _sc2(eidx, h2T, aaT)
    outT = _tc3(numer2, denom2, b2.reshape(_C2, 1))
    return outT[:, :_N].T
